# baseline (speedup 1.0000x reference)
import jax
import jax.numpy as jnp
from jax.experimental import pallas as pl
from jax.experimental.pallas import tpu as pltpu

_RB = 64
_NBUF = 8


def _body(x_hbm, out_ref, *scratch):
    bufs = scratch[:_NBUF]
    sems = scratch[_NBUF:]
    B = x_hbm.shape[0]
    n = B // _RB

    def mk(i):
        slot = i % _NBUF
        return pltpu.make_async_copy(
            x_hbm.at[pl.ds(i * _RB, _RB), :], bufs[slot], sems[slot]
        )

    for i in range(min(_NBUF, n)):
        mk(i).start()
    for i in range(n):
        mk(i).wait()
        out_ref[pl.ds(i * _RB, _RB), :] = bufs[i % _NBUF][:, :128] * 2.0
        if i + _NBUF < n:
            mk(i + _NBUF).start()


def kernel(x_seq, emb):
    B, K = x_seq.shape
    H = emb.shape[1]
    scratch = [pltpu.VMEM((_RB, K), jnp.float32) for _ in range(_NBUF)] + [
        pltpu.SemaphoreType.DMA for _ in range(_NBUF)
    ]
    return pl.pallas_call(
        _body,
        in_specs=[pl.BlockSpec(memory_space=pl.ANY)],
        out_specs=pl.BlockSpec(memory_space=pltpu.VMEM),
        out_shape=jax.ShapeDtypeStruct((B, H), jnp.float32),
        scratch_shapes=scratch,
    )(x_seq)


# speedup vs baseline: 3.2734x; 3.2734x over previous
import jax
import jax.numpy as jnp
from jax.experimental import pallas as pl

_KB = 2000


def _body(xT_ref, emb_ref, out_ref):
    p = jax.lax.dot_general(
        xT_ref[...].astype(jnp.bfloat16),
        emb_ref[...].astype(jnp.bfloat16),
        (((0,), (0,)), ((), ())),
        preferred_element_type=jnp.float32,
    )

    @pl.when(pl.program_id(0) == 0)
    def _():
        out_ref[...] = p

    @pl.when(pl.program_id(0) != 0)
    def _():
        out_ref[...] += p


def kernel(x_seq, emb):
    B, K = x_seq.shape
    H = emb.shape[1]
    return pl.pallas_call(
        _body,
        grid=(K // _KB,),
        in_specs=[
            pl.BlockSpec((_KB, B), lambda i: (i, 0)),
            pl.BlockSpec((_KB, H), lambda i: (i, 0)),
        ],
        out_specs=pl.BlockSpec((B, H), lambda i: (0, 0)),
        out_shape=jax.ShapeDtypeStruct((B, H), jnp.float32),
    )(x_seq.T, emb)
